# baseline (speedup 1.0000x reference)
import functools

import jax
import jax.numpy as jnp
from jax.experimental import pallas as pl
from jax.experimental.pallas import tpu as pltpu

_H_PAD = 56


def _num_tensorcores():
    try:
        n = jax.devices()[0].num_cores
        return int(n) if n else 1
    except Exception:
        return 2


@functools.partial(jax.jit, static_argnames=("block_b",))
def _forward(x, w1, b1, w2, b2, block_b=32768):
    B, S = x.shape
    H = w1.shape[0]
    A = w2.shape[0]

    w1 = w1.astype(jnp.float32)
    b1 = b1.astype(jnp.float32).reshape(-1)
    w2 = w2.astype(jnp.float32)
    b2 = b2.astype(jnp.float32).reshape(-1)

    w1p = jnp.zeros((_H_PAD, S), jnp.float32).at[:H, :].set(w1)
    b1p = jnp.zeros((_H_PAD, 1), jnp.float32).at[:H, 0].set(b1)
    w2p = jnp.zeros((A, _H_PAD), jnp.float32).at[:, :H].set(w2)
    b2p = b2[:, None]

    n_cores = _num_tensorcores()
    chunk = block_b * n_cores
    b_pad = -(-B // chunk) * chunk
    xt = jnp.zeros((S, b_pad), jnp.float32).at[:, :B].set(x.T)
    n_steps = b_pad // block_b

    mesh = pltpu.create_tensorcore_mesh("core", num_cores=n_cores)

    @pl.kernel(out_type=jax.ShapeDtypeStruct((A, b_pad), jnp.float32),
               mesh=mesh,
               scratch_types=[
                   pltpu.VMEM((_H_PAD, S), jnp.float32),
                   pltpu.VMEM((_H_PAD, 1), jnp.float32),
                   pltpu.VMEM((A, _H_PAD), jnp.float32),
                   pltpu.VMEM((A, 1), jnp.float32),
               ])
    def run(xt_ref, w1_ref, b1_ref, w2_ref, b2_ref, o_ref,
            w1_s, b1_s, w2_s, b2_s):
        pltpu.sync_copy(w1_ref, w1_s)
        pltpu.sync_copy(b1_ref, b1_s)
        pltpu.sync_copy(w2_ref, w2_s)
        pltpu.sync_copy(b2_ref, b2_s)

        def step(xt_blk, o_blk):
            ht = jnp.dot(w1_s[...], xt_blk[...],
                         preferred_element_type=jnp.float32)
            ht = jnp.tanh(ht + b1_s[...])
            o_blk[...] = (
                jnp.dot(w2_s[...], ht, preferred_element_type=jnp.float32)
                + b2_s[...])

        pltpu.emit_pipeline(
            step,
            grid=(n_steps,),
            in_specs=[pl.BlockSpec((S, block_b), lambda i: (0, i))],
            out_specs=[pl.BlockSpec((A, block_b), lambda i: (0, i))],
            core_axis_name="core",
            dimension_semantics=(pltpu.PARALLEL,),
        )(xt_ref, o_ref)

    yt = run(xt, w1p, b1p, w2p, b2p)
    return yt[:, :B].T


def kernel(x, w1, b1, w2, b2):
    return _forward(x, w1, b1, w2, b2)
